# baseline (device time: 20427 ns/iter reference)
import os

import jax
import jax.numpy as jnp
from jax import lax
from jax.experimental import pallas as pl
from jax.experimental.pallas import tpu as pltpu

N_DEV = 8
N_GLOBAL = 8192
EPS = 1e-5
CHUNKS = int(os.environ.get("CHUNKS", "2"))
NO_COMM = os.environ.get("NO_COMM") == "1"
SCOPES = os.environ.get("SCOPES") == "1"

import contextlib


def _scope(name):
    return jax.named_scope(name) if SCOPES else contextlib.nullcontext()


def kernel(x, gamma):
    m, n_per = x.shape
    m_c = m // CHUNKS
    sub = m_c // 128
    g2 = gamma.reshape(1, n_per)

    def body(x_hbm, g_ref, out_hbm, x_vmem, out_vmem, *scratch):
        comms = scratch[:CHUNKS]
        send_sems = scratch[CHUNKS : 2 * CHUNKS]
        recv_sems = scratch[2 * CHUNKS : 3 * CHUNKS]
        in_sems = scratch[3 * CHUNKS]
        out_sems = scratch[3 * CHUNKS + 1]
        my = lax.axis_index("i")

        if not NO_COMM:
            barrier_sem = pltpu.get_barrier_semaphore()
            for d in range(1, N_DEV):
                pl.semaphore_signal(
                    barrier_sem,
                    inc=1,
                    device_id=(lax.rem(my + d, N_DEV),),
                    device_id_type=pl.DeviceIdType.MESH,
                )

        in_copies = []
        for c in range(CHUNKS):
            cp = pltpu.make_async_copy(
                x_hbm.at[pl.ds(c * m_c, m_c), :],
                x_vmem.at[pl.ds(c * m_c, m_c), :],
                in_sems.at[c],
            )
            cp.start()
            in_copies.append(cp)

        ones_v = jnp.ones((1, n_per), jnp.float32)

        for c in range(CHUNKS):
            with _scope(f"in_wait#c={c}"):
                in_copies[c].wait()
            with _scope(f"partial#c={c}"):
                xc = x_vmem[pl.ds(c * m_c, m_c), :]
                sq = xc * xc
                part = lax.dot_general(
                    ones_v,
                    sq,
                    (((1,), (1,)), ((), ())),
                    preferred_element_type=jnp.float32,
                )
                packed = part.reshape(sub, 128)
                comms[c][my] = packed

            if not NO_COMM:
                if c == 0:
                    with _scope("barrier_wait"):
                        pl.semaphore_wait(barrier_sem, N_DEV - 1)
                with _scope(f"send#c={c}"):
                    for d in range(1, N_DEV):
                        rdma = pltpu.make_async_remote_copy(
                            src_ref=comms[c].at[my],
                            dst_ref=comms[c].at[my],
                            send_sem=send_sems[c].at[d],
                            recv_sem=recv_sems[c].at[my],
                            device_id=(lax.rem(my + d, N_DEV),),
                            device_id_type=pl.DeviceIdType.MESH,
                        )
                        rdma.start()

        out_copies = []
        for c in range(CHUNKS):
            if not NO_COMM:
                with _scope(f"recv_wait#c={c}"):
                    for d in range(1, N_DEV):
                        src = lax.rem(my + d, N_DEV)
                        recv = pltpu.make_async_remote_copy(
                            src_ref=comms[c].at[src],
                            dst_ref=comms[c].at[src],
                            send_sem=send_sems[c].at[d],
                            recv_sem=recv_sems[c].at[src],
                            device_id=(src,),
                            device_id_type=pl.DeviceIdType.MESH,
                        )
                        recv.wait_recv()

            with _scope(f"normalize#c={c}"):
                total = jnp.sum(comms[c][...], axis=0)
                inv = lax.rsqrt(total / N_GLOBAL + EPS)
                for i in range(sub):
                    inv_blk = inv[i : i + 1, :].reshape(128, 1)
                    r0 = c * m_c + i * 128
                    xb = x_vmem[pl.ds(r0, 128), :]
                    out_vmem[pl.ds(r0, 128), :] = g_ref[...] * (xb * inv_blk)

                cp = pltpu.make_async_copy(
                    out_vmem.at[pl.ds(c * m_c, m_c), :],
                    out_hbm.at[pl.ds(c * m_c, m_c), :],
                    out_sems.at[c],
                )
                cp.start()
                out_copies.append(cp)

        with _scope("drain"):
            for cp in out_copies:
                cp.wait()
            for c in range(CHUNKS if not NO_COMM else 0):
                for d in range(1, N_DEV):
                    send = pltpu.make_async_remote_copy(
                        src_ref=comms[c].at[my],
                        dst_ref=comms[c].at[my],
                        send_sem=send_sems[c].at[d],
                        recv_sem=recv_sems[c].at[my],
                        device_id=(lax.rem(my + d, N_DEV),),
                        device_id_type=pl.DeviceIdType.MESH,
                    )
                    send.wait_send()

    return pl.pallas_call(
        body,
        out_shape=pltpu.MemorySpace.HBM((m, n_per), jnp.float32),
        in_specs=[
            pl.BlockSpec(memory_space=pltpu.MemorySpace.HBM),
            pl.BlockSpec(memory_space=pltpu.VMEM),
        ],
        out_specs=pl.BlockSpec(memory_space=pltpu.MemorySpace.HBM),
        scratch_shapes=(
            [pltpu.VMEM((m, n_per), jnp.float32)]
            + [pltpu.VMEM((m, n_per), jnp.float32)]
            + [pltpu.VMEM((N_DEV, m // CHUNKS // 128, 128), jnp.float32)]
            * CHUNKS
            + [pltpu.SemaphoreType.DMA((N_DEV,))] * CHUNKS
            + [pltpu.SemaphoreType.DMA((N_DEV,))] * CHUNKS
            + [pltpu.SemaphoreType.DMA((CHUNKS,))]
            + [pltpu.SemaphoreType.DMA((CHUNKS,))]
        ),
        compiler_params=pltpu.CompilerParams(
            collective_id=None if NO_COMM else 0
        ),
    )(pltpu.with_memory_space_constraint(x, pltpu.MemorySpace.HBM), g2)


# device time: 14009 ns/iter; 1.4581x vs baseline; 1.4581x over previous
import os

import jax
import jax.numpy as jnp
from jax import lax
from jax.experimental import pallas as pl
from jax.experimental.pallas import tpu as pltpu

N_DEV = 8
N_GLOBAL = 8192
EPS = 1e-5
CHUNKS = int(os.environ.get("CHUNKS", "2"))
NO_COMM = os.environ.get("NO_COMM") == "1"


def kernel(x, gamma):
    m, n_per = x.shape
    m_c = m // CHUNKS
    sub = m_c // 128
    g2 = gamma.reshape(1, n_per)

    def body(x_hbm, g_hbm, out_ref, x_vmem, g_vmem, *scratch):
        comms = scratch[:CHUNKS]
        send_sems = scratch[CHUNKS : 2 * CHUNKS]
        recv_sems = scratch[2 * CHUNKS : 3 * CHUNKS]
        in_sems = scratch[3 * CHUNKS]
        g_sem = scratch[3 * CHUNKS + 1]
        my = lax.axis_index("i")

        if not NO_COMM:
            barrier_sem = pltpu.get_barrier_semaphore()
            for d in range(1, N_DEV):
                pl.semaphore_signal(
                    barrier_sem,
                    inc=1,
                    device_id=(lax.rem(my + d, N_DEV),),
                    device_id_type=pl.DeviceIdType.MESH,
                )

        g_copy = pltpu.make_async_copy(g_hbm, g_vmem, g_sem)
        g_copy.start()
        in_copies = []
        for c in range(CHUNKS):
            cp = pltpu.make_async_copy(
                x_hbm.at[pl.ds(c * m_c, m_c), :],
                x_vmem.at[pl.ds(c * m_c, m_c), :],
                in_sems.at[c],
            )
            cp.start()
            in_copies.append(cp)

        ones_v = jnp.ones((1, n_per), jnp.float32)

        for c in range(CHUNKS):
            in_copies[c].wait()
            xc = x_vmem[pl.ds(c * m_c, m_c), :]
            sq = xc * xc
            part = lax.dot_general(
                ones_v,
                sq,
                (((1,), (1,)), ((), ())),
                preferred_element_type=jnp.float32,
            )
            packed = part.reshape(sub, 128)
            comms[c][my] = packed

            if not NO_COMM:
                if c == 0:
                    pl.semaphore_wait(barrier_sem, N_DEV - 1)
                for d in range(1, N_DEV):
                    rdma = pltpu.make_async_remote_copy(
                        src_ref=comms[c].at[my],
                        dst_ref=comms[c].at[my],
                        send_sem=send_sems[c].at[d],
                        recv_sem=recv_sems[c].at[my],
                        device_id=(lax.rem(my + d, N_DEV),),
                        device_id_type=pl.DeviceIdType.MESH,
                    )
                    rdma.start()

        g_copy.wait()

        for c in range(CHUNKS):
            if not NO_COMM:
                for d in range(1, N_DEV):
                    src = lax.rem(my + d, N_DEV)
                    recv = pltpu.make_async_remote_copy(
                        src_ref=comms[c].at[src],
                        dst_ref=comms[c].at[src],
                        send_sem=send_sems[c].at[d],
                        recv_sem=recv_sems[c].at[src],
                        device_id=(src,),
                        device_id_type=pl.DeviceIdType.MESH,
                    )
                    recv.wait_recv()

            total = jnp.sum(comms[c][...], axis=0)
            inv = lax.rsqrt(total / N_GLOBAL + EPS)
            for i in range(sub):
                inv_blk = inv[i : i + 1, :].reshape(128, 1)
                r0 = c * m_c + i * 128
                xb = x_vmem[pl.ds(r0, 128), :]
                out_ref[pl.ds(r0, 128), :] = g_vmem[...] * (xb * inv_blk)

        for c in range(CHUNKS if not NO_COMM else 0):
            for d in range(1, N_DEV):
                send = pltpu.make_async_remote_copy(
                    src_ref=comms[c].at[my],
                    dst_ref=comms[c].at[my],
                    send_sem=send_sems[c].at[d],
                    recv_sem=recv_sems[c].at[my],
                    device_id=(lax.rem(my + d, N_DEV),),
                    device_id_type=pl.DeviceIdType.MESH,
                )
                send.wait_send()

    return pl.pallas_call(
        body,
        out_shape=jax.ShapeDtypeStruct((m, n_per), jnp.float32),
        in_specs=[
            pl.BlockSpec(memory_space=pltpu.MemorySpace.HBM),
            pl.BlockSpec(memory_space=pltpu.MemorySpace.HBM),
        ],
        out_specs=pl.BlockSpec(memory_space=pltpu.VMEM),
        scratch_shapes=(
            [pltpu.VMEM((m, n_per), jnp.float32)]
            + [pltpu.VMEM((1, n_per), jnp.float32)]
            + [pltpu.VMEM((N_DEV, m // CHUNKS // 128, 128), jnp.float32)]
            * CHUNKS
            + [pltpu.SemaphoreType.DMA((N_DEV,))] * CHUNKS
            + [pltpu.SemaphoreType.DMA((N_DEV,))] * CHUNKS
            + [pltpu.SemaphoreType.DMA((CHUNKS,))]
            + [pltpu.SemaphoreType.DMA]
        ),
        compiler_params=pltpu.CompilerParams(
            collective_id=None if NO_COMM else 0
        ),
    )(
        pltpu.with_memory_space_constraint(x, pltpu.MemorySpace.HBM),
        pltpu.with_memory_space_constraint(g2, pltpu.MemorySpace.HBM),
    )


# device time: 13932 ns/iter; 1.4662x vs baseline; 1.0055x over previous
import os

import jax
import jax.numpy as jnp
from jax import lax
from jax.experimental import pallas as pl
from jax.experimental.pallas import tpu as pltpu

N_DEV = 8
N_GLOBAL = 8192
EPS = 1e-5
CHUNKS = int(os.environ.get("CHUNKS", "4"))
NO_COMM = os.environ.get("NO_COMM") == "1"


def kernel(x, gamma):
    m, n_per = x.shape
    m_c = m // CHUNKS
    sub = m_c // 128
    g2 = gamma.reshape(1, n_per)

    def body(x_hbm, g_hbm, out_ref, x_vmem, g_vmem, *scratch):
        comms = scratch[:CHUNKS]
        send_sems = scratch[CHUNKS : 2 * CHUNKS]
        recv_sems = scratch[2 * CHUNKS : 3 * CHUNKS]
        in_sems = scratch[3 * CHUNKS]
        g_sem = scratch[3 * CHUNKS + 1]
        my = lax.axis_index("i")

        if not NO_COMM:
            barrier_sem = pltpu.get_barrier_semaphore()
            for d in range(1, N_DEV):
                pl.semaphore_signal(
                    barrier_sem,
                    inc=1,
                    device_id=(lax.rem(my + d, N_DEV),),
                    device_id_type=pl.DeviceIdType.MESH,
                )

        g_copy = pltpu.make_async_copy(g_hbm, g_vmem, g_sem)
        g_copy.start()
        in_copies = []
        for c in range(CHUNKS):
            cp = pltpu.make_async_copy(
                x_hbm.at[pl.ds(c * m_c, m_c), :],
                x_vmem.at[pl.ds(c * m_c, m_c), :],
                in_sems.at[c],
            )
            cp.start()
            in_copies.append(cp)

        ones_v = jnp.ones((1, n_per), jnp.float32)

        for c in range(CHUNKS):
            in_copies[c].wait()
            xc = x_vmem[pl.ds(c * m_c, m_c), :]
            sq = xc * xc
            part = lax.dot_general(
                ones_v,
                sq,
                (((1,), (1,)), ((), ())),
                preferred_element_type=jnp.float32,
            )
            packed = part.reshape(sub, 128)
            comms[c][my] = packed

            if not NO_COMM:
                if c == 0:
                    pl.semaphore_wait(barrier_sem, N_DEV - 1)
                for d in range(1, N_DEV):
                    rdma = pltpu.make_async_remote_copy(
                        src_ref=comms[c].at[my],
                        dst_ref=comms[c].at[my],
                        send_sem=send_sems[c].at[d],
                        recv_sem=recv_sems[c].at[my],
                        device_id=(lax.rem(my + d, N_DEV),),
                        device_id_type=pl.DeviceIdType.MESH,
                    )
                    rdma.start()

        g_copy.wait()

        for c in range(CHUNKS):
            if not NO_COMM:
                for d in range(1, N_DEV):
                    src = lax.rem(my + d, N_DEV)
                    recv = pltpu.make_async_remote_copy(
                        src_ref=comms[c].at[src],
                        dst_ref=comms[c].at[src],
                        send_sem=send_sems[c].at[d],
                        recv_sem=recv_sems[c].at[src],
                        device_id=(src,),
                        device_id_type=pl.DeviceIdType.MESH,
                    )
                    recv.wait_recv()

            total = jnp.sum(comms[c][...], axis=0)
            inv = lax.rsqrt(total / N_GLOBAL + EPS)
            for i in range(sub):
                inv_blk = inv[i : i + 1, :].reshape(128, 1)
                r0 = c * m_c + i * 128
                xb = x_vmem[pl.ds(r0, 128), :]
                out_ref[pl.ds(r0, 128), :] = g_vmem[...] * (xb * inv_blk)

        for c in range(CHUNKS if not NO_COMM else 0):
            for d in range(1, N_DEV):
                send = pltpu.make_async_remote_copy(
                    src_ref=comms[c].at[my],
                    dst_ref=comms[c].at[my],
                    send_sem=send_sems[c].at[d],
                    recv_sem=recv_sems[c].at[my],
                    device_id=(lax.rem(my + d, N_DEV),),
                    device_id_type=pl.DeviceIdType.MESH,
                )
                send.wait_send()

    return pl.pallas_call(
        body,
        out_shape=jax.ShapeDtypeStruct((m, n_per), jnp.float32),
        in_specs=[
            pl.BlockSpec(memory_space=pltpu.MemorySpace.HBM),
            pl.BlockSpec(memory_space=pltpu.MemorySpace.HBM),
        ],
        out_specs=pl.BlockSpec(memory_space=pltpu.VMEM),
        scratch_shapes=(
            [pltpu.VMEM((m, n_per), jnp.float32)]
            + [pltpu.VMEM((1, n_per), jnp.float32)]
            + [pltpu.VMEM((N_DEV, m // CHUNKS // 128, 128), jnp.float32)]
            * CHUNKS
            + [pltpu.SemaphoreType.DMA((N_DEV,))] * CHUNKS
            + [pltpu.SemaphoreType.DMA((N_DEV,))] * CHUNKS
            + [pltpu.SemaphoreType.DMA((CHUNKS,))]
            + [pltpu.SemaphoreType.DMA]
        ),
        compiler_params=pltpu.CompilerParams(
            collective_id=None if NO_COMM else 0
        ),
    )(
        pltpu.with_memory_space_constraint(x, pltpu.MemorySpace.HBM),
        pltpu.with_memory_space_constraint(g2, pltpu.MemorySpace.HBM),
    )
